# baseline (speedup 1.0000x reference)
import functools

import jax
import jax.numpy as jnp
from jax import lax
from jax.experimental import pallas as pl
from jax.experimental.pallas import tpu as pltpu
from jax.experimental.pallas import tpu_sc as plsc

N_NODES = 10000
N_EDGES = 320000
D = 128

NC = 2
NS = 16
NW = NC * NS

CHUNK = 128
CH_PER_W = 80
NCHUNK = NW * CH_PER_W
E_PAD = NCHUNK * CHUNK
NBUF = 2
N_PHASE = 2
K_PH = CH_PER_W // N_PHASE
N_ACC = N_NODES + 8
ROWS_PER_SUB = 624
TAIL_ROWS = N_NODES - NS * ROWS_PER_SUB


def _sc_aggregate(x, src2d, dst2d, zeros):
    mesh = plsc.VectorSubcoreMesh(core_axis_name="c", subcore_axis_name="s")

    @functools.partial(
        pl.kernel,
        mesh=mesh,
        out_type=jax.ShapeDtypeStruct((NC, N_NODES, D), jnp.float32),
        scratch_types=[
            pltpu.VMEM((K_PH, CHUNK), jnp.int32),
            pltpu.VMEM((K_PH, CHUNK), jnp.int32),
            pltpu.VMEM((NBUF, CHUNK, D), jnp.float32),
            pltpu.VMEM_SHARED((N_ACC, D), jnp.float32),
            [pltpu.SemaphoreType.DMA] * NBUF,
        ],
    )
    def agg(x_hbm, src_hbm, dst_hbm, zeros_hbm, out_hbm,
            src_v, dst_v, rows_v, acc, sems):
        c = lax.axis_index("c")
        s = lax.axis_index("s")
        wid = s * NC + c

        row0 = s * ROWS_PER_SUB
        pltpu.sync_copy(zeros_hbm.at[pl.ds(row0, ROWS_PER_SUB)],
                        acc.at[pl.ds(row0, ROWS_PER_SUB)])

        @pl.when(s == NS - 1)
        def _():
            t0 = NS * ROWS_PER_SUB
            pltpu.sync_copy(zeros_hbm.at[pl.ds(t0, TAIL_ROWS)],
                            acc.at[pl.ds(t0, TAIL_ROWS)])

        plsc.subcore_barrier()

        c0 = wid * CH_PER_W
        for p in range(N_PHASE):
            pltpu.sync_copy(src_hbm.at[pl.ds(c0 + p * K_PH, K_PH)], src_v)
            pltpu.sync_copy(dst_hbm.at[pl.ds(c0 + p * K_PH, K_PH)], dst_v)

            for b in range(NBUF):
                pltpu.async_copy(x_hbm.at[src_v.at[b]], rows_v.at[b], sems[b])

            def body(i, carry):
                for b in range(NBUF):
                    j = NBUF * i + b
                    pltpu.make_async_copy(x_hbm.at[src_v.at[j]],
                                          rows_v.at[b], sems[b]).wait()
                    pltpu.async_copy(rows_v.at[b], acc.at[dst_v.at[j]],
                                     sems[b], add=True).wait()

                    @pl.when(j + NBUF < K_PH)
                    def _():
                        pltpu.async_copy(x_hbm.at[src_v.at[j + NBUF]],
                                         rows_v.at[b], sems[b])

                return carry

            lax.fori_loop(0, K_PH // NBUF, body, 0)

        plsc.subcore_barrier()

        pltpu.sync_copy(acc.at[pl.ds(row0, ROWS_PER_SUB)],
                        out_hbm.at[c, pl.ds(row0, ROWS_PER_SUB)])

        @pl.when(s == NS - 1)
        def _():
            t0 = NS * ROWS_PER_SUB
            pltpu.sync_copy(acc.at[pl.ds(t0, TAIL_ROWS)],
                            out_hbm.at[c, pl.ds(t0, TAIL_ROWS)])

    return agg(x, src2d, dst2d, zeros)


def _mlp_block(x_ref, p0_ref, p1_ref, w1_ref, b1_ref, w2_ref, b2_ref, o_ref):
    h = x_ref[...] + p0_ref[...] + p1_ref[...]
    h = jnp.dot(h, w1_ref[...], preferred_element_type=jnp.float32) + b1_ref[...]
    h = jnp.maximum(h, 0.0)
    o_ref[...] = (
        jnp.dot(h, w2_ref[...], preferred_element_type=jnp.float32) + b2_ref[...]
    )


def _tc_mlp(x, p0, p1, W1, b1, W2, b2):
    blk = 1000
    grid = (N_NODES // blk,)
    row_spec = pl.BlockSpec((blk, D), lambda i: (i, 0))
    full_spec = pl.BlockSpec((D, D), lambda i: (0, 0))
    bias_spec = pl.BlockSpec((1, D), lambda i: (0, 0))
    return pl.pallas_call(
        _mlp_block,
        grid=grid,
        in_specs=[row_spec, row_spec, row_spec,
                  full_spec, bias_spec, full_spec, bias_spec],
        out_specs=row_spec,
        out_shape=jax.ShapeDtypeStruct((N_NODES, D), jnp.float32),
    )(x, p0, p1, W1.T, b1.reshape(1, D), W2.T, b2.reshape(1, D))


def kernel(x, edge_index, W1, b1, W2, b2):
    src = edge_index[0].astype(jnp.int32)
    dst = edge_index[1].astype(jnp.int32)
    pad = E_PAD - N_EDGES
    src2d = jnp.concatenate([src, jnp.zeros((pad,), jnp.int32)]).reshape(
        NCHUNK, CHUNK)
    dst2d = jnp.concatenate(
        [dst, jnp.full((pad,), N_NODES, jnp.int32)]).reshape(NCHUNK, CHUNK)
    zeros = jnp.zeros((N_NODES, D), jnp.float32)
    partials = _sc_aggregate(x, src2d, dst2d, zeros)
    return _tc_mlp(x, partials[0], partials[1], W1, b1, W2, b2)


# speedup vs baseline: 3.1155x; 3.1155x over previous
import functools

import jax
import jax.numpy as jnp
from jax import lax
from jax.experimental import pallas as pl
from jax.experimental.pallas import tpu as pltpu
from jax.experimental.pallas import tpu_sc as plsc

N_NODES = 10000
N_EDGES = 320000
D = 128

NC = 2
NS = 16
NW = NC * NS

CHUNK = 128
CH_PER_W = 80
NCHUNK = NW * CH_PER_W
E_PAD = NCHUNK * CHUNK
NBUF = 2
N_PHASE = 2
K_PH = CH_PER_W // N_PHASE
N_REAL_CHUNK = N_EDGES // CHUNK
ROWS_PER_SUB = 624
TAIL_ROWS = N_NODES - NS * ROWS_PER_SUB


def _sc_aggregate(x, src2d, dst2d, zeros):
    mesh = plsc.VectorSubcoreMesh(core_axis_name="c", subcore_axis_name="s")

    @functools.partial(
        pl.kernel,
        mesh=mesh,
        out_type=jax.ShapeDtypeStruct((NC, N_NODES, D), jnp.float32),
        scratch_types=[
            pltpu.VMEM((K_PH, CHUNK), jnp.int32),
            pltpu.VMEM((K_PH, CHUNK), jnp.int32),
            pltpu.VMEM((NBUF, CHUNK, D), jnp.float32),
            pltpu.VMEM_SHARED((N_NODES, D), jnp.float32),
            [pltpu.SemaphoreType.DMA] * NBUF,
        ],
    )
    def agg(x_hbm, src_hbm, dst_hbm, zeros_hbm, out_hbm,
            src_v, dst_v, rows_v, acc, sems):
        c = lax.axis_index("c")
        s = lax.axis_index("s")
        wid = s * NC + c

        row0 = s * ROWS_PER_SUB
        pltpu.sync_copy(zeros_hbm.at[pl.ds(row0, ROWS_PER_SUB)],
                        acc.at[pl.ds(row0, ROWS_PER_SUB)])

        @pl.when(s == NS - 1)
        def _():
            t0 = NS * ROWS_PER_SUB
            pltpu.sync_copy(zeros_hbm.at[pl.ds(t0, TAIL_ROWS)],
                            acc.at[pl.ds(t0, TAIL_ROWS)])

        plsc.subcore_barrier()

        c0 = wid * CH_PER_W
        for p in range(N_PHASE):
            pltpu.sync_copy(src_hbm.at[pl.ds(c0 + p * K_PH, K_PH)], src_v)
            pltpu.sync_copy(dst_hbm.at[pl.ds(c0 + p * K_PH, K_PH)], dst_v)

            ph0 = c0 + p * K_PH

            for b in range(NBUF):
                @pl.when(ph0 + b < N_REAL_CHUNK)
                def _():
                    pltpu.async_copy(x_hbm.at[src_v.at[b]],
                                     rows_v.at[b], sems[b])

            def body(i, carry):
                for b in range(NBUF):
                    j = NBUF * i + b

                    @pl.when(ph0 + j < N_REAL_CHUNK)
                    def _():
                        pltpu.make_async_copy(x_hbm.at[src_v.at[j]],
                                              rows_v.at[b], sems[b]).wait()
                        pltpu.async_copy(rows_v.at[b], acc.at[dst_v.at[j]],
                                         sems[b], add=True).wait()

                        @pl.when(ph0 + j + NBUF < N_REAL_CHUNK)
                        def _():
                            pltpu.async_copy(x_hbm.at[src_v.at[j + NBUF]],
                                             rows_v.at[b], sems[b])

                return carry

            lax.fori_loop(0, K_PH // NBUF - 1, body, 0)

            for b in range(NBUF):
                j = K_PH - NBUF + b

                @pl.when(ph0 + j < N_REAL_CHUNK)
                def _():
                    pltpu.make_async_copy(x_hbm.at[src_v.at[j]],
                                          rows_v.at[b], sems[b]).wait()
                    pltpu.async_copy(rows_v.at[b], acc.at[dst_v.at[j]],
                                     sems[b], add=True).wait()

        plsc.subcore_barrier()

        pltpu.sync_copy(acc.at[pl.ds(row0, ROWS_PER_SUB)],
                        out_hbm.at[c, pl.ds(row0, ROWS_PER_SUB)])

        @pl.when(s == NS - 1)
        def _():
            t0 = NS * ROWS_PER_SUB
            pltpu.sync_copy(acc.at[pl.ds(t0, TAIL_ROWS)],
                            out_hbm.at[c, pl.ds(t0, TAIL_ROWS)])

    return agg(x, src2d, dst2d, zeros)


def _mlp_block(x_ref, p0_ref, p1_ref, w1_ref, b1_ref, w2_ref, b2_ref, o_ref):
    h = x_ref[...] + p0_ref[...] + p1_ref[...]
    h = jnp.dot(h, w1_ref[...], preferred_element_type=jnp.float32) + b1_ref[...]
    h = jnp.maximum(h, 0.0)
    o_ref[...] = (
        jnp.dot(h, w2_ref[...], preferred_element_type=jnp.float32) + b2_ref[...]
    )


def _tc_mlp(x, p0, p1, W1, b1, W2, b2):
    blk = 1000
    grid = (N_NODES // blk,)
    row_spec = pl.BlockSpec((blk, D), lambda i: (i, 0))
    full_spec = pl.BlockSpec((D, D), lambda i: (0, 0))
    bias_spec = pl.BlockSpec((1, D), lambda i: (0, 0))
    return pl.pallas_call(
        _mlp_block,
        grid=grid,
        in_specs=[row_spec, row_spec, row_spec,
                  full_spec, bias_spec, full_spec, bias_spec],
        out_specs=row_spec,
        out_shape=jax.ShapeDtypeStruct((N_NODES, D), jnp.float32),
    )(x, p0, p1, W1.T, b1.reshape(1, D), W2.T, b2.reshape(1, D))


def kernel(x, edge_index, W1, b1, W2, b2):
    src = edge_index[0].astype(jnp.int32)
    dst = edge_index[1].astype(jnp.int32)
    pad = E_PAD - N_EDGES
    src2d = jnp.concatenate([src, jnp.zeros((pad,), jnp.int32)]).reshape(
        NCHUNK, CHUNK)
    dst2d = jnp.concatenate(
        [dst, jnp.zeros((pad,), jnp.int32)]).reshape(NCHUNK, CHUNK)
    zeros = jnp.zeros((N_NODES, D), jnp.float32)
    partials = _sc_aggregate(x, src2d, dst2d, zeros)
    return _tc_mlp(x, partials[0], partials[1], W1, b1, W2, b2)
